# baseline (speedup 1.0000x reference)
import functools

import jax
import jax.numpy as jnp
from jax.experimental import pallas as pl
from jax.experimental.pallas import tpu as pltpu


def _stage(j, mb_ref, qn_ref, acc_ref, read_buf, write_buf):
    s = jax.lax.dot_general(
        read_buf[...], qn_ref[...],
        (((1,), (0,)), ((), ())),
        preferred_element_type=jnp.float32)
    bm = jnp.max(s, axis=0, keepdims=True)
    acc_ref[...] = jnp.where(j == 1, bm, jnp.maximum(acc_ref[...], bm))

    mb = mb_ref[...]
    ksq = jnp.sum(mb * mb, axis=1, keepdims=True)
    r = 1.0 / (jnp.sqrt(ksq) + 1e-12)
    write_buf[...] = (mb * r).astype(jnp.bfloat16)


def _knn_body(B, C, HW, BK, nsteps, qf_ref, mb_ref, out_ref,
              qn_ref, qsq_ref, acc_ref, buf0, buf1):
    Q = B * HW
    j = pl.program_id(0)

    @pl.when(j == 0)
    def _init():
        for b in range(B):
            f = qf_ref[b * C:(b + 1) * C, :]
            nrm = jnp.sqrt(jnp.sum(f * f, axis=0, keepdims=True))
            qn = f / (nrm + 1e-12)
            qn_ref[:, b * HW:(b + 1) * HW] = qn.astype(jnp.bfloat16)
            qsq_ref[0:1, b * HW:(b + 1) * HW] = jnp.sum(
                qn * qn, axis=0, keepdims=True)

    @pl.when(j % 2 == 0)
    def _even():
        _stage(j, mb_ref, qn_ref, acc_ref, buf1, buf0)

    @pl.when(j % 2 == 1)
    def _odd():
        _stage(j, mb_ref, qn_ref, acc_ref, buf0, buf1)

    @pl.when(j == nsteps)
    def _finish():
        d2 = qsq_ref[...] + 1.0 - 2.0 * acc_ref[...]
        d2b = jnp.broadcast_to(d2, (B, Q))
        col = jax.lax.broadcasted_iota(jnp.int32, (B, Q), 1)
        row = jax.lax.broadcasted_iota(jnp.int32, (B, Q), 0)
        masked = jnp.where(col // HW == row, d2b, -jnp.inf)
        out_ref[...] = jnp.max(masked, axis=1, keepdims=True)


def kernel(features, memory_bank):
    B, C, H, W = features.shape
    K, _ = memory_bank.shape
    HW = H * W
    Q = B * HW
    BK = 1000
    nsteps = K // BK
    qf = features.reshape(B * C, HW)

    out = pl.pallas_call(
        functools.partial(_knn_body, B, C, HW, BK, nsteps),
        grid=(nsteps + 1,),
        in_specs=[
            pl.BlockSpec((B * C, HW), lambda j: (0, 0)),
            pl.BlockSpec((BK, C), lambda j: (jnp.minimum(j, nsteps - 1), 0)),
        ],
        out_specs=pl.BlockSpec((B, 1), lambda j: (0, 0)),
        out_shape=jax.ShapeDtypeStruct((B, 1), jnp.float32),
        scratch_shapes=[
            pltpu.VMEM((C, Q), jnp.bfloat16),
            pltpu.VMEM((1, Q), jnp.float32),
            pltpu.VMEM((1, Q), jnp.float32),
            pltpu.VMEM((BK, C), jnp.bfloat16),
            pltpu.VMEM((BK, C), jnp.bfloat16),
        ],
        compiler_params=pltpu.CompilerParams(
            dimension_semantics=("arbitrary",)),
    )(qf, memory_bank)
    return out.reshape(B)


# speedup vs baseline: 1.0690x; 1.0690x over previous
import functools

import jax
import jax.numpy as jnp
from jax.experimental import pallas as pl
from jax.experimental.pallas import tpu as pltpu


def _knn_body(B, C, HW, BK, CH, nsteps, qf_ref, mb_ref, out_ref,
              qn_ref, qsq_ref, acc_ref, mbn_ref):
    Q = B * HW
    CB = BK // CH
    j = pl.program_id(0)

    @pl.when(j == 0)
    def _init():
        for b in range(B):
            f = qf_ref[b * C:(b + 1) * C, :]
            fsq = jnp.sum(f * f, axis=0, keepdims=True)
            rq = 1.0 / (jnp.sqrt(fsq) + 1e-12)
            qn_ref[:, b * HW:(b + 1) * HW] = (f * rq).astype(jnp.bfloat16)
            qsq_ref[0:1, b * HW:(b + 1) * HW] = fsq * rq * rq

    def norm_chunk(c):
        mbc = mb_ref[c * CB:(c + 1) * CB, :]
        ksq = jnp.sum(mbc * mbc, axis=1, keepdims=True)
        rr = 1.0 / (jnp.sqrt(ksq) + 1e-12)
        mbn_ref[c * CB:(c + 1) * CB, :] = (mbc * rr).astype(jnp.bfloat16)

    def mm_chunk(c):
        s = jax.lax.dot_general(
            mbn_ref[c * CB:(c + 1) * CB, :], qn_ref[...],
            (((1,), (0,)), ((), ())),
            preferred_element_type=jnp.float32)
        return jnp.max(s, axis=0, keepdims=True)

    norm_chunk(0)
    bm = None
    for c in range(CH):
        if c + 1 < CH:
            norm_chunk(c + 1)
        m = mm_chunk(c)
        bm = m if bm is None else jnp.maximum(bm, m)
    acc_ref[...] = jnp.where(j == 0, bm, jnp.maximum(acc_ref[...], bm))

    @pl.when(j == nsteps - 1)
    def _finish():
        d2 = qsq_ref[...] + 1.0 - 2.0 * acc_ref[...]
        d2b = jnp.broadcast_to(d2, (B, Q))
        col = jax.lax.broadcasted_iota(jnp.int32, (B, Q), 1)
        row = jax.lax.broadcasted_iota(jnp.int32, (B, Q), 0)
        masked = jnp.where(col // HW == row, d2b, -jnp.inf)
        out_ref[...] = jnp.max(masked, axis=1, keepdims=True)


def kernel(features, memory_bank):
    B, C, H, W = features.shape
    K, _ = memory_bank.shape
    HW = H * W
    Q = B * HW
    BK = 1000
    CH = 5
    nsteps = K // BK
    qf = features.reshape(B * C, HW)

    out = pl.pallas_call(
        functools.partial(_knn_body, B, C, HW, BK, CH, nsteps),
        grid=(nsteps,),
        in_specs=[
            pl.BlockSpec((B * C, HW), lambda j: (0, 0)),
            pl.BlockSpec((BK, C), lambda j: (j, 0)),
        ],
        out_specs=pl.BlockSpec((B, 1), lambda j: (0, 0)),
        out_shape=jax.ShapeDtypeStruct((B, 1), jnp.float32),
        scratch_shapes=[
            pltpu.VMEM((C, Q), jnp.bfloat16),
            pltpu.VMEM((1, Q), jnp.float32),
            pltpu.VMEM((1, Q), jnp.float32),
            pltpu.VMEM((BK, C), jnp.bfloat16),
        ],
        compiler_params=pltpu.CompilerParams(
            dimension_semantics=("arbitrary",)),
    )(qf, memory_bank)
    return out.reshape(B)
